# Initial kernel scaffold; baseline (speedup 1.0000x reference)
import jax
import jax.numpy as jnp
from jax.experimental import pallas as pl


def kernel(pc):
    raise NotImplementedError("write your pallas kernel here")


# speedup vs baseline: 357.2639x; 357.2639x over previous
import functools

import jax
import jax.numpy as jnp
from jax import lax
from jax.experimental import pallas as pl
from jax.experimental.pallas import tpu as pltpu
from jax.experimental.pallas import tpu_sc as plsc

HEIGHT = 512
WIDTH = 512
INTENSITY = 255.0
NBINS = HEIGHT * WIDTH
HIST_WORDS = 2 * NBINS

N_POINTS = 2_000_000
CHUNK = 3200
KROWS = CHUNK // 128
NCHUNKS = N_POINTS // CHUNK
NC, NS = 2, 16
NW = NC * NS

MM_BLOCK = 10_000
MM_GRID = N_POINTS // MM_BLOCK

STRIPE = HIST_WORDS // NS
DUMP = NBINS // NS


def _minmax_body(pc_ref, o_ref):
    i = pl.program_id(0)
    d = pc_ref[:]
    xmn = jnp.min(d[:, 0:1])
    xmx = jnp.max(d[:, 0:1])
    ymn = jnp.min(d[:, 1:2])
    ymx = jnp.max(d[:, 1:2])
    zmn = jnp.min(d[:, 2:3])
    zmx = jnp.max(d[:, 2:3])
    rows = lax.broadcasted_iota(jnp.int32, (8, 128), 0)
    cur = jnp.where(rows == 0, xmn,
          jnp.where(rows == 1, xmx,
          jnp.where(rows == 2, ymn,
          jnp.where(rows == 3, ymx,
          jnp.where(rows == 4, zmn, zmx)))))
    is_min = (rows % 2) == 0
    acc = o_ref[:]
    comb = jnp.where(is_min, jnp.minimum(acc, cur), jnp.maximum(acc, cur))
    o_ref[:] = jnp.where(i == 0, cur, comb)


_minmax_call = pl.pallas_call(
    _minmax_body,
    grid=(MM_GRID,),
    in_specs=[pl.BlockSpec((MM_BLOCK, 3), lambda i: (i, 0))],
    out_specs=pl.BlockSpec((8, 128), lambda i: (0, 0)),
    out_shape=jax.ShapeDtypeStruct((8, 128), jnp.float32),
)


def _combine_body(p_ref, o_ref):
    s = p_ref[0, 0] + p_ref[1, 0]
    c = p_ref[0, 1] + p_ref[1, 1]
    o_ref[:] = jnp.where(c > 0, s / c, 0.0)


_combine_call = pl.pallas_call(
    _combine_body,
    grid=(8,),
    in_specs=[pl.BlockSpec((2, 2, 64, 512), lambda r: (0, 0, r, 0))],
    out_specs=pl.BlockSpec((64, 512), lambda r: (r, 0)),
    out_shape=jax.ShapeDtypeStruct((HEIGHT, WIDTH), jnp.float32),
)


def _sc_body(pc_ref, mm_ref, out_ref,
             pcbuf, mmbuf, idxs, idxc, zss, ones, obuf, hist):
    cid = lax.axis_index("c")
    sid = lax.axis_index("s")
    wid = sid * NC + cid

    def zero_obuf(t, _):
        obuf[pl.ds(t * 16, 16)] = jnp.zeros((16,), jnp.float32)
        return 0
    lax.fori_loop(0, DUMP // 16, zero_obuf, 0)
    pltpu.sync_copy(obuf, hist.at[pl.ds(sid * STRIPE, DUMP)])
    pltpu.sync_copy(obuf, hist.at[pl.ds(sid * STRIPE + DUMP, DUMP)])

    pltpu.sync_copy(mm_ref, mmbuf)
    xmin = mmbuf[pl.ds(0, 16)]
    xmax = mmbuf[pl.ds(128, 16)]
    ymin = mmbuf[pl.ds(256, 16)]
    ymax = mmbuf[pl.ds(384, 16)]
    zmin = mmbuf[pl.ds(512, 16)]
    zmax = mmbuf[pl.ds(640, 16)]
    sx = (WIDTH - 1.0) / (xmax - xmin)
    sy = (HEIGHT - 1.0) / (ymax - ymin)
    sz = INTENSITY / (zmax - zmin)

    ii = lax.iota(jnp.int32, 16)
    i3 = ii * 3

    def init_ones(t, _):
        ones[pl.ds(t * 16, 16)] = jnp.full((16,), 1.0, jnp.float32)
        return 0
    lax.fori_loop(0, CHUNK // 16, init_ones, 0)

    plsc.subcore_barrier()

    nch = jnp.where(wid < NCHUNKS % NW, NCHUNKS // NW + 1, NCHUNKS // NW)

    def chunk_body(t, _):
        g = wid + t * NW
        base3 = g * (CHUNK * 3)
        pltpu.sync_copy(pc_ref.at[pl.ds(base3, CHUNK * 3)], pcbuf)

        def row_body(j, _):
            o3 = j * 384
            for u in range(8):
                xi = (o3 + u * 48) + i3
                xv = plsc.load_gather(pcbuf, [xi])
                yv = plsc.load_gather(pcbuf, [xi + 1])
                zv = plsc.load_gather(pcbuf, [xi + 2])
                xb = ((xv - xmin) * sx).astype(jnp.int32)
                yb = ((yv - ymin) * sy).astype(jnp.int32)
                idx = (511 - yb) * 512 + xb
                zs = (zv - zmin) * sz
                o = j * 128 + u * 16
                idxs[pl.ds(o, 16)] = idx
                idxc[pl.ds(o, 16)] = idx + NBINS
                zss[pl.ds(o, 16)] = zs
            return 0
        lax.fori_loop(0, KROWS, row_body, 0)

        pltpu.sync_copy(zss, hist.at[idxs], add=True)
        pltpu.sync_copy(ones, hist.at[idxc], add=True)
        return 0
    lax.fori_loop(0, nch, chunk_body, 0)

    plsc.subcore_barrier()

    pltpu.sync_copy(hist.at[pl.ds(sid * DUMP, DUMP)], obuf)
    pltpu.sync_copy(obuf, out_ref.at[cid, 0, pl.ds(sid * DUMP, DUMP)])
    pltpu.sync_copy(hist.at[pl.ds(NBINS + sid * DUMP, DUMP)], obuf)
    pltpu.sync_copy(obuf, out_ref.at[cid, 1, pl.ds(sid * DUMP, DUMP)])


_sc_call = pl.kernel(
    _sc_body,
    out_type=jax.ShapeDtypeStruct((NC, 2, NBINS), jnp.float32),
    mesh=plsc.VectorSubcoreMesh(core_axis_name="c", subcore_axis_name="s"),
    scratch_types=[
        pltpu.VMEM((CHUNK * 3,), jnp.float32),
        pltpu.VMEM((1024,), jnp.float32),
        pltpu.VMEM((CHUNK,), jnp.int32),
        pltpu.VMEM((CHUNK,), jnp.int32),
        pltpu.VMEM((CHUNK,), jnp.float32),
        pltpu.VMEM((CHUNK,), jnp.float32),
        pltpu.VMEM((DUMP,), jnp.float32),
        pltpu.VMEM_SHARED((HIST_WORDS,), jnp.float32),
    ],
    compiler_params=pltpu.CompilerParams(needs_layout_passes=False),
)


@jax.jit
def kernel(pc):
    mm = _minmax_call(pc)
    parts = _sc_call(pc.reshape(-1), mm.reshape(-1))
    return _combine_call(parts.reshape(NC, 2, HEIGHT, WIDTH))


# speedup vs baseline: 389.5810x; 1.0905x over previous
import functools

import jax
import jax.numpy as jnp
from jax import lax
from jax.experimental import pallas as pl
from jax.experimental.pallas import tpu as pltpu
from jax.experimental.pallas import tpu_sc as plsc

HEIGHT = 512
WIDTH = 512
INTENSITY = 255.0
NBINS = HEIGHT * WIDTH
HIST_WORDS = 2 * NBINS

N_POINTS = 2_000_000
CHUNK = 3200
KROWS = CHUNK // 128
NCHUNKS = N_POINTS // CHUNK
NC, NS = 2, 16
NW = NC * NS

MM_BLOCK = 20_000
MM_GRID = N_POINTS // MM_BLOCK

STRIPE = HIST_WORDS // NS
DUMP = NBINS // NS


def _minmax_body(pc_ref, o_ref):
    i = pl.program_id(0)
    d = pc_ref[:]
    dmin = jnp.broadcast_to(jnp.min(d, axis=0, keepdims=True), (4, 3))
    dmax = jnp.broadcast_to(jnp.max(d, axis=0, keepdims=True), (4, 3))
    cur = jnp.concatenate(
        [jnp.concatenate([dmin, dmax], axis=0),
         jnp.zeros((8, 125), jnp.float32)], axis=1)
    acc = o_ref[:]
    comb = jnp.concatenate(
        [jnp.minimum(acc[:4], cur[:4]), jnp.maximum(acc[4:], cur[4:])], axis=0)
    o_ref[:] = jnp.where(i == 0, cur, comb)


_minmax_call = pl.pallas_call(
    _minmax_body,
    grid=(MM_GRID,),
    in_specs=[pl.BlockSpec((MM_BLOCK, 3), lambda i: (i, 0))],
    out_specs=pl.BlockSpec((8, 128), lambda i: (0, 0)),
    out_shape=jax.ShapeDtypeStruct((8, 128), jnp.float32),
)


def _combine_body(p_ref, o_ref):
    s = p_ref[0, 0] + p_ref[1, 0]
    c = p_ref[0, 1] + p_ref[1, 1]
    o_ref[:] = jnp.where(c > 0, s / c, 0.0)


_combine_call = pl.pallas_call(
    _combine_body,
    grid=(8,),
    in_specs=[pl.BlockSpec((2, 2, 64, 512), lambda r: (0, 0, r, 0))],
    out_specs=pl.BlockSpec((64, 512), lambda r: (r, 0)),
    out_shape=jax.ShapeDtypeStruct((HEIGHT, WIDTH), jnp.float32),
)


def _sc_body(pc_ref, mm_ref, out_ref,
             pcbuf, mmbuf, idxs, idxc, zss, ones, obuf, hist):
    cid = lax.axis_index("c")
    sid = lax.axis_index("s")
    wid = sid * NC + cid

    def zero_obuf(t, _):
        obuf[pl.ds(t * 16, 16)] = jnp.zeros((16,), jnp.float32)
        return 0
    lax.fori_loop(0, DUMP // 16, zero_obuf, 0)
    pltpu.sync_copy(obuf, hist.at[pl.ds(sid * STRIPE, DUMP)])
    pltpu.sync_copy(obuf, hist.at[pl.ds(sid * STRIPE + DUMP, DUMP)])

    pltpu.sync_copy(mm_ref, mmbuf)
    xmin = mmbuf[pl.ds(0, 16)]
    xmax = mmbuf[pl.ds(128, 16)]
    ymin = mmbuf[pl.ds(256, 16)]
    ymax = mmbuf[pl.ds(384, 16)]
    zmin = mmbuf[pl.ds(512, 16)]
    zmax = mmbuf[pl.ds(640, 16)]
    sx = (WIDTH - 1.0) / (xmax - xmin)
    sy = (HEIGHT - 1.0) / (ymax - ymin)
    sz = INTENSITY / (zmax - zmin)

    ii = lax.iota(jnp.int32, 16)

    def init_ones(t, _):
        ones[pl.ds(t * 16, 16)] = jnp.full((16,), 1.0, jnp.float32)
        return 0
    lax.fori_loop(0, CHUNK // 16, init_ones, 0)

    plsc.subcore_barrier()

    nch = jnp.where(wid < NCHUNKS % NW, NCHUNKS // NW + 1, NCHUNKS // NW)

    i3 = ii * 3

    def chunk_body(t, _):
        g = wid + t * NW
        pltpu.sync_copy(pc_ref.at[pl.ds(g * CHUNK * 3, CHUNK * 3)], pcbuf)

        def row_body(j, _):
            o128 = j * 128
            for u in range(8):
                xi = (o128 + u * 16) * 3 + i3
                xv = plsc.load_gather(pcbuf, [xi])
                yv = plsc.load_gather(pcbuf, [xi + 1])
                zv = plsc.load_gather(pcbuf, [xi + 2])
                xb = ((xv - xmin) * sx).astype(jnp.int32)
                yb = ((yv - ymin) * sy).astype(jnp.int32)
                idx = (511 - yb) * 512 + xb
                idx = jnp.minimum(jnp.maximum(idx, 0), NBINS - 1)
                zs = (zv - zmin) * sz
                o = o128 + u * 16
                idxs[pl.ds(o, 16)] = idx
                idxc[pl.ds(o, 16)] = idx + NBINS
                zss[pl.ds(o, 16)] = zs
            return 0
        lax.fori_loop(0, KROWS, row_body, 0)

        pltpu.sync_copy(zss, hist.at[idxs], add=True)
        pltpu.sync_copy(ones, hist.at[idxc], add=True)
        return 0
    lax.fori_loop(0, nch, chunk_body, 0)

    plsc.subcore_barrier()

    pltpu.sync_copy(hist.at[pl.ds(sid * DUMP, DUMP)], obuf)
    pltpu.sync_copy(obuf, out_ref.at[cid, 0, pl.ds(sid * DUMP, DUMP)])
    pltpu.sync_copy(hist.at[pl.ds(NBINS + sid * DUMP, DUMP)], obuf)
    pltpu.sync_copy(obuf, out_ref.at[cid, 1, pl.ds(sid * DUMP, DUMP)])


_sc_call = pl.kernel(
    _sc_body,
    out_type=jax.ShapeDtypeStruct((NC, 2, NBINS), jnp.float32),
    mesh=plsc.VectorSubcoreMesh(core_axis_name="c", subcore_axis_name="s",
                                num_cores=NC, num_subcores=NS),
    scratch_types=[
        pltpu.VMEM((CHUNK * 3,), jnp.float32),
        pltpu.VMEM((1024,), jnp.float32),
        pltpu.VMEM((CHUNK,), jnp.int32),
        pltpu.VMEM((CHUNK,), jnp.int32),
        pltpu.VMEM((CHUNK,), jnp.float32),
        pltpu.VMEM((CHUNK,), jnp.float32),
        pltpu.VMEM((DUMP,), jnp.float32),
        pltpu.VMEM_SHARED((HIST_WORDS,), jnp.float32),
    ],
    compiler_params=pltpu.CompilerParams(needs_layout_passes=False),
)


@jax.jit
def kernel(pc):
    mm = _minmax_call(pc)
    vals = jnp.stack([mm[0, 0], mm[4, 0], mm[0, 1], mm[4, 1],
                      mm[0, 2], mm[4, 2],
                      jnp.float32(0.0), jnp.float32(0.0)])
    mm1024 = jnp.broadcast_to(vals[:, None], (8, 128)).reshape(-1)
    parts = _sc_call(pc.reshape(-1), mm1024)
    return _combine_call(parts.reshape(NC, 2, HEIGHT, WIDTH))


# speedup vs baseline: 644.5350x; 1.6544x over previous
import functools

import jax
import jax.numpy as jnp
from jax import lax
from jax.experimental import pallas as pl
from jax.experimental.pallas import tpu as pltpu
from jax.experimental.pallas import tpu_sc as plsc

HEIGHT = 512
WIDTH = 512
INTENSITY = 255.0
NBINS = HEIGHT * WIDTH
HIST_WORDS = 2 * NBINS

N_POINTS = 2_000_000
CHUNK = 3200
KROWS = CHUNK // 128
NCHUNKS = N_POINTS // CHUNK
NC, NS = 2, 16
NW = NC * NS

MM_BLOCK = 20_480
MM_GRID = -(-N_POINTS // MM_BLOCK)

STRIPE = HIST_WORDS // NS
DUMP = NBINS // NS


def _minmax_body(pc_ref, o_ref, fl_ref):
    i = pl.program_id(0)
    d = pc_ref[:]
    d3 = jnp.transpose(d.reshape(MM_BLOCK // 128, 128, 3), (0, 2, 1))
    fl_ref[:] = jnp.concatenate(
        [d3, jnp.zeros((MM_BLOCK // 128, 5, 128), jnp.float32)], axis=1)
    rid = lax.broadcasted_iota(jnp.int32, (MM_BLOCK, 1), 0)
    valid = (i * MM_BLOCK + rid) < N_POINTS
    dlo = jnp.where(valid, d, jnp.inf)
    dhi = jnp.where(valid, d, -jnp.inf)
    dmin = jnp.broadcast_to(jnp.min(dlo, axis=0, keepdims=True), (4, 3))
    dmax = jnp.broadcast_to(jnp.max(dhi, axis=0, keepdims=True), (4, 3))
    cur = jnp.concatenate(
        [jnp.concatenate([dmin, dmax], axis=0),
         jnp.zeros((8, 125), jnp.float32)], axis=1)
    acc = o_ref[:]
    comb = jnp.concatenate(
        [jnp.minimum(acc[:4], cur[:4]), jnp.maximum(acc[4:], cur[4:])], axis=0)
    o_ref[:] = jnp.where(i == 0, cur, comb)


_minmax_call = pl.pallas_call(
    _minmax_body,
    grid=(MM_GRID,),
    in_specs=[pl.BlockSpec((MM_BLOCK, 3), lambda i: (i, 0))],
    out_specs=[pl.BlockSpec((8, 128), lambda i: (0, 0)),
               pl.BlockSpec((MM_BLOCK // 128, 8, 128), lambda i: (i, 0, 0))],
    out_shape=[jax.ShapeDtypeStruct((8, 128), jnp.float32),
               jax.ShapeDtypeStruct(
                   (MM_GRID * MM_BLOCK // 128, 8, 128), jnp.float32)],
)


def _combine_body(p_ref, o_ref):
    s = p_ref[0, 0] + p_ref[1, 0]
    c = p_ref[0, 1] + p_ref[1, 1]
    o_ref[:] = jnp.where(c > 0, s / c, 0.0)


_combine_call = pl.pallas_call(
    _combine_body,
    grid=(8,),
    in_specs=[pl.BlockSpec((2, 2, 64, 512), lambda r: (0, 0, r, 0))],
    out_specs=pl.BlockSpec((64, 512), lambda r: (r, 0)),
    out_shape=jax.ShapeDtypeStruct((HEIGHT, WIDTH), jnp.float32),
)


def _sc_body(pc_ref, mm_ref, out_ref,
             pcbuf, mmbuf, idxs, idxc, zss, ones, obuf, hist):
    cid = lax.axis_index("c")
    sid = lax.axis_index("s")
    wid = sid * NC + cid

    def zero_obuf(t, _):
        obuf[pl.ds(t * 16, 16)] = jnp.zeros((16,), jnp.float32)
        return 0
    lax.fori_loop(0, DUMP // 16, zero_obuf, 0)
    pltpu.sync_copy(obuf, hist.at[pl.ds(sid * STRIPE, DUMP)])
    pltpu.sync_copy(obuf, hist.at[pl.ds(sid * STRIPE + DUMP, DUMP)])

    pltpu.sync_copy(mm_ref, mmbuf)
    xmin = mmbuf[pl.ds(0, 16)]
    xmax = mmbuf[pl.ds(128, 16)]
    ymin = mmbuf[pl.ds(256, 16)]
    ymax = mmbuf[pl.ds(384, 16)]
    zmin = mmbuf[pl.ds(512, 16)]
    zmax = mmbuf[pl.ds(640, 16)]
    sx = (WIDTH - 1.0) / (xmax - xmin)
    sy = (HEIGHT - 1.0) / (ymax - ymin)
    sz = INTENSITY / (zmax - zmin)

    def init_ones(t, _):
        ones[pl.ds(t * 16, 16)] = jnp.full((16,), 1.0, jnp.float32)
        return 0
    lax.fori_loop(0, CHUNK // 16, init_ones, 0)

    plsc.subcore_barrier()

    nch = jnp.where(wid < NCHUNKS % NW, NCHUNKS // NW + 1, NCHUNKS // NW)

    def chunk_body(t, _):
        g = wid + t * NW
        pltpu.sync_copy(pc_ref.at[pl.ds(g * KROWS, KROWS)], pcbuf)

        def row_body(j, _):
            o128 = j * 128
            for u in range(8):
                xv = pcbuf[j, 0, pl.ds(u * 16, 16)]
                yv = pcbuf[j, 1, pl.ds(u * 16, 16)]
                zv = pcbuf[j, 2, pl.ds(u * 16, 16)]
                xb = ((xv - xmin) * sx).astype(jnp.int32)
                yb = ((yv - ymin) * sy).astype(jnp.int32)
                idx = (511 - yb) * 512 + xb
                idx = jnp.minimum(jnp.maximum(idx, 0), NBINS - 1)
                zs = (zv - zmin) * sz
                o = o128 + u * 16
                idxs[pl.ds(o, 16)] = idx
                idxc[pl.ds(o, 16)] = idx + NBINS
                zss[pl.ds(o, 16)] = zs
            return 0
        lax.fori_loop(0, KROWS, row_body, 0)

        pltpu.sync_copy(zss, hist.at[idxs], add=True)
        pltpu.sync_copy(ones, hist.at[idxc], add=True)
        return 0
    lax.fori_loop(0, nch, chunk_body, 0)

    plsc.subcore_barrier()

    pltpu.sync_copy(hist.at[pl.ds(sid * DUMP, DUMP)], obuf)
    pltpu.sync_copy(obuf, out_ref.at[cid, 0, pl.ds(sid * DUMP, DUMP)])
    pltpu.sync_copy(hist.at[pl.ds(NBINS + sid * DUMP, DUMP)], obuf)
    pltpu.sync_copy(obuf, out_ref.at[cid, 1, pl.ds(sid * DUMP, DUMP)])


_sc_call = pl.kernel(
    _sc_body,
    out_type=jax.ShapeDtypeStruct((NC, 2, NBINS), jnp.float32),
    mesh=plsc.VectorSubcoreMesh(core_axis_name="c", subcore_axis_name="s",
                                num_cores=NC, num_subcores=NS),
    scratch_types=[
        pltpu.VMEM((KROWS, 8, 128), jnp.float32),
        pltpu.VMEM((1024,), jnp.float32),
        pltpu.VMEM((CHUNK,), jnp.int32),
        pltpu.VMEM((CHUNK,), jnp.int32),
        pltpu.VMEM((CHUNK,), jnp.float32),
        pltpu.VMEM((CHUNK,), jnp.float32),
        pltpu.VMEM((DUMP,), jnp.float32),
        pltpu.VMEM_SHARED((HIST_WORDS,), jnp.float32),
    ],
    compiler_params=pltpu.CompilerParams(needs_layout_passes=False),
)


@jax.jit
def kernel(pc):
    mm, pcf = _minmax_call(pc)
    vals = jnp.stack([mm[0, 0], mm[4, 0], mm[0, 1], mm[4, 1],
                      mm[0, 2], mm[4, 2],
                      jnp.float32(0.0), jnp.float32(0.0)])
    mm1024 = jnp.broadcast_to(vals[:, None], (8, 128)).reshape(-1)
    parts = _sc_call(pcf, mm1024)
    return _combine_call(parts.reshape(NC, 2, HEIGHT, WIDTH))


# speedup vs baseline: 2475.6285x; 3.8410x over previous
import functools

import jax
import jax.numpy as jnp
from jax import lax
from jax.experimental import pallas as pl
from jax.experimental.pallas import tpu as pltpu
from jax.experimental.pallas import tpu_sc as plsc

HEIGHT = 512
WIDTH = 512
INTENSITY = 255.0
NBINS = HEIGHT * WIDTH
HIST_WORDS = 2 * NBINS

N_POINTS = 2_000_000
CHUNK = 3200
KROWS = CHUNK // 128
NCHUNKS = N_POINTS // CHUNK
NC, NS = 2, 16
NW = NC * NS

MM_BLOCK = 20_480
MM_GRID = -(-N_POINTS // MM_BLOCK)

STRIPE = HIST_WORDS // NS
DUMP = NBINS // NS


def _minmax_body(pc_ref, o_ref, fl_ref):
    i = pl.program_id(0)
    d = pc_ref[:]
    d8 = jnp.concatenate(
        [d, jnp.zeros((5, MM_BLOCK), jnp.float32)], axis=0)
    fl_ref[:] = jnp.transpose(
        d8.reshape(8, MM_BLOCK // 128, 128), (1, 0, 2))
    cid = lax.broadcasted_iota(jnp.int32, (3, MM_BLOCK), 1)
    valid = (i * MM_BLOCK + cid) < N_POINTS
    dlo = jnp.where(valid, d, jnp.inf)
    dhi = jnp.where(valid, d, -jnp.inf)
    dmin = jnp.broadcast_to(jnp.min(dlo, axis=1, keepdims=True), (3, 128))
    dmax = jnp.broadcast_to(jnp.max(dhi, axis=1, keepdims=True), (3, 128))
    pad1 = jnp.zeros((1, 128), jnp.float32)
    cur = jnp.concatenate([dmin, pad1, dmax, pad1], axis=0)
    rows = lax.broadcasted_iota(jnp.int32, (8, 128), 0)
    acc = o_ref[:]
    comb = jnp.where(rows < 4, jnp.minimum(acc, cur), jnp.maximum(acc, cur))
    o_ref[:] = jnp.where(i == 0, cur, comb)


_minmax_call = pl.pallas_call(
    _minmax_body,
    grid=(MM_GRID,),
    in_specs=[pl.BlockSpec((3, MM_BLOCK), lambda i: (0, i))],
    out_specs=[pl.BlockSpec((8, 128), lambda i: (0, 0)),
               pl.BlockSpec((MM_BLOCK // 128, 8, 128), lambda i: (i, 0, 0))],
    out_shape=[jax.ShapeDtypeStruct((8, 128), jnp.float32),
               jax.ShapeDtypeStruct(
                   (MM_GRID * MM_BLOCK // 128, 8, 128), jnp.float32)],
)


def _combine_body(p_ref, o_ref):
    s = p_ref[0, 0] + p_ref[1, 0]
    c = p_ref[0, 1] + p_ref[1, 1]
    o_ref[:] = jnp.where(c > 0, s / c, 0.0)


_combine_call = pl.pallas_call(
    _combine_body,
    grid=(8,),
    in_specs=[pl.BlockSpec((2, 2, 64, 512), lambda r: (0, 0, r, 0))],
    out_specs=pl.BlockSpec((64, 512), lambda r: (r, 0)),
    out_shape=jax.ShapeDtypeStruct((HEIGHT, WIDTH), jnp.float32),
)


def _sc_body(pc_ref, mm_ref, out_ref,
             pcbuf, mmbuf, idxs, idxc, zss, ones, obuf, hist):
    cid = lax.axis_index("c")
    sid = lax.axis_index("s")
    wid = sid * NC + cid

    def zero_obuf(t, _):
        obuf[pl.ds(t * 16, 16)] = jnp.zeros((16,), jnp.float32)
        return 0
    lax.fori_loop(0, DUMP // 16, zero_obuf, 0)
    pltpu.sync_copy(obuf, hist.at[pl.ds(sid * STRIPE, DUMP)])
    pltpu.sync_copy(obuf, hist.at[pl.ds(sid * STRIPE + DUMP, DUMP)])

    pltpu.sync_copy(mm_ref, mmbuf)
    xmin = mmbuf[pl.ds(0, 16)]
    ymin = mmbuf[pl.ds(128, 16)]
    zmin = mmbuf[pl.ds(256, 16)]
    xmax = mmbuf[pl.ds(512, 16)]
    ymax = mmbuf[pl.ds(640, 16)]
    zmax = mmbuf[pl.ds(768, 16)]
    sx = (WIDTH - 1.0) / (xmax - xmin)
    sy = (HEIGHT - 1.0) / (ymax - ymin)
    sz = INTENSITY / (zmax - zmin)

    def init_ones(t, _):
        ones[pl.ds(t * 16, 16)] = jnp.full((16,), 1.0, jnp.float32)
        return 0
    lax.fori_loop(0, CHUNK // 16, init_ones, 0)

    plsc.subcore_barrier()

    nch = jnp.where(wid < NCHUNKS % NW, NCHUNKS // NW + 1, NCHUNKS // NW)

    def chunk_body(t, _):
        g = wid + t * NW
        pltpu.sync_copy(pc_ref.at[pl.ds(g * KROWS, KROWS)], pcbuf)

        def row_body(j, _):
            o128 = j * 128
            for u in range(8):
                xv = pcbuf[j, 0, pl.ds(u * 16, 16)]
                yv = pcbuf[j, 1, pl.ds(u * 16, 16)]
                zv = pcbuf[j, 2, pl.ds(u * 16, 16)]
                xb = ((xv - xmin) * sx).astype(jnp.int32)
                yb = ((yv - ymin) * sy).astype(jnp.int32)
                idx = (511 - yb) * 512 + xb
                idx = jnp.minimum(jnp.maximum(idx, 0), NBINS - 1)
                zs = (zv - zmin) * sz
                o = o128 + u * 16
                idxs[pl.ds(o, 16)] = idx
                idxc[pl.ds(o, 16)] = idx + NBINS
                zss[pl.ds(o, 16)] = zs
            return 0
        lax.fori_loop(0, KROWS, row_body, 0)

        pltpu.sync_copy(zss, hist.at[idxs], add=True)
        pltpu.sync_copy(ones, hist.at[idxc], add=True)
        return 0
    lax.fori_loop(0, nch, chunk_body, 0)

    plsc.subcore_barrier()

    pltpu.sync_copy(hist.at[pl.ds(sid * DUMP, DUMP)], obuf)
    pltpu.sync_copy(obuf, out_ref.at[cid, 0, pl.ds(sid * DUMP, DUMP)])
    pltpu.sync_copy(hist.at[pl.ds(NBINS + sid * DUMP, DUMP)], obuf)
    pltpu.sync_copy(obuf, out_ref.at[cid, 1, pl.ds(sid * DUMP, DUMP)])


_sc_call = pl.kernel(
    _sc_body,
    out_type=jax.ShapeDtypeStruct((NC, 2, NBINS), jnp.float32),
    mesh=plsc.VectorSubcoreMesh(core_axis_name="c", subcore_axis_name="s",
                                num_cores=NC, num_subcores=NS),
    scratch_types=[
        pltpu.VMEM((KROWS, 8, 128), jnp.float32),
        pltpu.VMEM((1024,), jnp.float32),
        pltpu.VMEM((CHUNK,), jnp.int32),
        pltpu.VMEM((CHUNK,), jnp.int32),
        pltpu.VMEM((CHUNK,), jnp.float32),
        pltpu.VMEM((CHUNK,), jnp.float32),
        pltpu.VMEM((DUMP,), jnp.float32),
        pltpu.VMEM_SHARED((HIST_WORDS,), jnp.float32),
    ],
    compiler_params=pltpu.CompilerParams(needs_layout_passes=False),
)


@jax.jit
def kernel(pc):
    mm, pcf = _minmax_call(pc.T)
    parts = _sc_call(pcf, mm.reshape(-1))
    return _combine_call(parts.reshape(NC, 2, HEIGHT, WIDTH))


# speedup vs baseline: 2884.4536x; 1.1651x over previous
import functools

import jax
import jax.numpy as jnp
from jax import lax
from jax.experimental import pallas as pl
from jax.experimental.pallas import tpu as pltpu
from jax.experimental.pallas import tpu_sc as plsc

HEIGHT = 512
WIDTH = 512
INTENSITY = 255.0
NBINS = HEIGHT * WIDTH
HIST_WORDS = 2 * NBINS

N_POINTS = 2_000_000
CHUNK = 3200
KROWS = CHUNK // 128
NCHUNKS = N_POINTS // CHUNK
NC, NS = 2, 16
NW = NC * NS

MM_BLOCK = 20_480
MM_GRID = -(-N_POINTS // MM_BLOCK)

STRIPE = HIST_WORDS // NS
DUMP = NBINS // NS


def _minmax_body(pc_ref, o_ref, fl_ref):
    i = pl.program_id(0)
    d = pc_ref[:]
    d8 = jnp.concatenate(
        [d, jnp.zeros((5, MM_BLOCK), jnp.float32)], axis=0)
    fl_ref[:] = jnp.transpose(
        d8.reshape(8, MM_BLOCK // 128, 128), (1, 0, 2))
    cid = lax.broadcasted_iota(jnp.int32, (3, MM_BLOCK), 1)
    valid = (i * MM_BLOCK + cid) < N_POINTS
    dlo = jnp.where(valid, d, jnp.inf)
    dhi = jnp.where(valid, d, -jnp.inf)
    dmin = jnp.broadcast_to(jnp.min(dlo, axis=1, keepdims=True), (3, 128))
    dmax = jnp.broadcast_to(jnp.max(dhi, axis=1, keepdims=True), (3, 128))
    pad1 = jnp.zeros((1, 128), jnp.float32)
    cur = jnp.concatenate([dmin, pad1, dmax, pad1], axis=0)
    rows = lax.broadcasted_iota(jnp.int32, (8, 128), 0)
    acc = o_ref[:]
    comb = jnp.where(rows < 4, jnp.minimum(acc, cur), jnp.maximum(acc, cur))
    o_ref[:] = jnp.where(i == 0, cur, comb)


_minmax_call = pl.pallas_call(
    _minmax_body,
    grid=(MM_GRID,),
    in_specs=[pl.BlockSpec((3, MM_BLOCK), lambda i: (0, i))],
    out_specs=[pl.BlockSpec((8, 128), lambda i: (0, 0)),
               pl.BlockSpec((MM_BLOCK // 128, 8, 128), lambda i: (i, 0, 0))],
    out_shape=[jax.ShapeDtypeStruct((8, 128), jnp.float32),
               jax.ShapeDtypeStruct(
                   (MM_GRID * MM_BLOCK // 128, 8, 128), jnp.float32)],
)


def _combine_body(p_ref, o_ref):
    s = p_ref[0, 0] + p_ref[1, 0]
    c = p_ref[0, 1] + p_ref[1, 1]
    o_ref[:] = jnp.where(c > 0, s / c, 0.0)


_combine_call = pl.pallas_call(
    _combine_body,
    grid=(8,),
    in_specs=[pl.BlockSpec((2, 2, 64, 512), lambda r: (0, 0, r, 0))],
    out_specs=pl.BlockSpec((64, 512), lambda r: (r, 0)),
    out_shape=jax.ShapeDtypeStruct((HEIGHT, WIDTH), jnp.float32),
)


NPAIRS = (NCHUNKS // NW + 1 + 1) // 2


def _sc_body(pc_ref, mm_ref, out_ref,
             pcbuf, mmbuf, idxsa, idxca, zssa, idxsb, idxcb, zssb,
             ones, obuf, hist, sema, semb):
    cid = lax.axis_index("c")
    sid = lax.axis_index("s")
    wid = sid * NC + cid

    def zero_obuf(t, _):
        obuf[pl.ds(t * 16, 16)] = jnp.zeros((16,), jnp.float32)
        return 0
    lax.fori_loop(0, DUMP // 16, zero_obuf, 0)
    pltpu.sync_copy(obuf, hist.at[pl.ds(sid * STRIPE, DUMP)])
    pltpu.sync_copy(obuf, hist.at[pl.ds(sid * STRIPE + DUMP, DUMP)])

    pltpu.sync_copy(mm_ref, mmbuf)
    xmin = mmbuf[pl.ds(0, 16)]
    ymin = mmbuf[pl.ds(128, 16)]
    zmin = mmbuf[pl.ds(256, 16)]
    xmax = mmbuf[pl.ds(512, 16)]
    ymax = mmbuf[pl.ds(640, 16)]
    zmax = mmbuf[pl.ds(768, 16)]
    sx = (WIDTH - 1.0) / (xmax - xmin)
    sy = (HEIGHT - 1.0) / (ymax - ymin)
    sz = INTENSITY / (zmax - zmin)

    def init_ones(t, _):
        ones[pl.ds(t * 16, 16)] = jnp.full((16,), 1.0, jnp.float32)
        return 0
    lax.fori_loop(0, CHUNK // 16, init_ones, 0)

    plsc.subcore_barrier()

    dummy = out_ref.at[cid, 0, pl.ds(0, CHUNK)]

    def compute_chunk(g, idxs, idxc, zss):
        pltpu.sync_copy(pc_ref.at[pl.ds(g * KROWS, KROWS)], pcbuf)

        def row_body(j, _):
            o128 = j * 128
            for u in range(8):
                xv = pcbuf[j, 0, pl.ds(u * 16, 16)]
                yv = pcbuf[j, 1, pl.ds(u * 16, 16)]
                zv = pcbuf[j, 2, pl.ds(u * 16, 16)]
                xb = ((xv - xmin) * sx).astype(jnp.int32)
                yb = ((yv - ymin) * sy).astype(jnp.int32)
                idx = (511 - yb) * 512 + xb
                idx = jnp.minimum(jnp.maximum(idx, 0), NBINS - 1)
                zs = (zv - zmin) * sz
                o = o128 + u * 16
                idxs[pl.ds(o, 16)] = idx
                idxc[pl.ds(o, 16)] = idx + NBINS
                zss[pl.ds(o, 16)] = zs
            return 0
        lax.fori_loop(0, KROWS, row_body, 0)

    def pair_body(p, _):
        g0 = wid + (2 * p) * NW
        g1 = g0 + NW

        @pl.when(g0 < NCHUNKS)
        def _():
            @pl.when(p > 0)
            def _():
                pltpu.make_async_copy(dummy, zssa, sema).wait()
                pltpu.make_async_copy(dummy, zssa, sema).wait()
            compute_chunk(g0, idxsa, idxca, zssa)
            pltpu.async_copy(zssa, hist.at[idxsa], sema, add=True)
            pltpu.async_copy(ones, hist.at[idxca], sema, add=True)

        @pl.when(g1 < NCHUNKS)
        def _():
            @pl.when(p > 0)
            def _():
                pltpu.make_async_copy(dummy, zssb, semb).wait()
                pltpu.make_async_copy(dummy, zssb, semb).wait()
            compute_chunk(g1, idxsb, idxcb, zssb)
            pltpu.async_copy(zssb, hist.at[idxsb], semb, add=True)
            pltpu.async_copy(ones, hist.at[idxcb], semb, add=True)
        return 0
    lax.fori_loop(0, NPAIRS, pair_body, 0)

    pltpu.make_async_copy(dummy, zssa, sema).wait()
    pltpu.make_async_copy(dummy, zssa, sema).wait()
    pltpu.make_async_copy(dummy, zssb, semb).wait()
    pltpu.make_async_copy(dummy, zssb, semb).wait()

    plsc.subcore_barrier()

    pltpu.sync_copy(hist.at[pl.ds(sid * DUMP, DUMP)], obuf)
    pltpu.sync_copy(obuf, out_ref.at[cid, 0, pl.ds(sid * DUMP, DUMP)])
    pltpu.sync_copy(hist.at[pl.ds(NBINS + sid * DUMP, DUMP)], obuf)
    pltpu.sync_copy(obuf, out_ref.at[cid, 1, pl.ds(sid * DUMP, DUMP)])


_sc_call = pl.kernel(
    _sc_body,
    out_type=jax.ShapeDtypeStruct((NC, 2, NBINS), jnp.float32),
    mesh=plsc.VectorSubcoreMesh(core_axis_name="c", subcore_axis_name="s",
                                num_cores=NC, num_subcores=NS),
    scratch_types=[
        pltpu.VMEM((KROWS, 8, 128), jnp.float32),
        pltpu.VMEM((1024,), jnp.float32),
        pltpu.VMEM((CHUNK,), jnp.int32),
        pltpu.VMEM((CHUNK,), jnp.int32),
        pltpu.VMEM((CHUNK,), jnp.float32),
        pltpu.VMEM((CHUNK,), jnp.int32),
        pltpu.VMEM((CHUNK,), jnp.int32),
        pltpu.VMEM((CHUNK,), jnp.float32),
        pltpu.VMEM((CHUNK,), jnp.float32),
        pltpu.VMEM((DUMP,), jnp.float32),
        pltpu.VMEM_SHARED((HIST_WORDS,), jnp.float32),
        pltpu.SemaphoreType.DMA,
        pltpu.SemaphoreType.DMA,
    ],
    compiler_params=pltpu.CompilerParams(needs_layout_passes=False),
)


@jax.jit
def kernel(pc):
    mm, pcf = _minmax_call(pc.T)
    parts = _sc_call(pcf, mm.reshape(-1))
    return _combine_call(parts.reshape(NC, 2, HEIGHT, WIDTH))
